# baseline (speedup 1.0000x reference)
import functools
import jax
import jax.numpy as jnp
from jax import lax
from jax.experimental import pallas as pl
from jax.experimental.pallas import tpu as pltpu
from jax.experimental.pallas import tpu_sc as plsc

D = 64
F = 26
NC = 2
NS = 16
NW = NC * NS
NB = 16
NBUF = 2


def _emb_body(idx_hbm, table_hbm, out_hbm, idx_v, rows_v, gsem0, gsem1,
              osem0, osem1):
    wid = lax.axis_index("s") * NC + lax.axis_index("c")
    b_per_w = out_hbm.shape[0] // NW
    nchunk = b_per_w // NB
    gsems = [gsem0, gsem1]
    osems = [osem0, osem1]

    def gather_cp(b0, buf, j):
        return pltpu.make_async_copy(
            table_hbm.at[idx_v.at[buf].at[j]],
            rows_v.at[buf].at[j],
            gsems[buf],
        )

    def out_cp(b0, buf):
        return pltpu.make_async_copy(
            rows_v.at[buf],
            out_hbm.at[pl.ds(b0, NB)],
            osems[buf],
        )

    def fire_gather(b0, buf):
        pltpu.sync_copy(idx_hbm.at[pl.ds(b0, NB)], idx_v.at[buf])
        for j in range(NB):
            gather_cp(b0, buf, j).start()

    def wait_gather(b0, buf):
        for j in range(NB):
            gather_cp(b0, buf, j).wait()

    base = wid * b_per_w
    fire_gather(base, 0)
    fire_gather(base + NB, 1)

    def step(g, carry):
        for buf in range(NBUF):
            b0 = base + (NBUF * g + buf) * NB
            wait_gather(b0, buf)
            out_cp(b0, buf).start()
        for buf in range(NBUF):
            b0 = base + (NBUF * g + buf) * NB
            out_cp(b0, buf).wait()

            @pl.when(g < nchunk // NBUF - 1)
            def _():
                fire_gather(b0 + NBUF * NB, buf)

        return carry

    lax.fori_loop(0, nchunk // NBUF, step, 0)


def kernel(input, weight):
    B, F_ = input.shape
    idx = input.astype(jnp.int32)

    gather = functools.partial(
        pl.kernel,
        mesh=plsc.VectorSubcoreMesh(core_axis_name="c", subcore_axis_name="s"),
        out_type=jax.ShapeDtypeStruct((B, F_, D), jnp.float32),
        scratch_types=[
            pltpu.VMEM((NBUF, NB, F_), jnp.int32),
            pltpu.VMEM((NBUF, NB, F_, D), jnp.float32),
            pltpu.SemaphoreType.DMA,
            pltpu.SemaphoreType.DMA,
            pltpu.SemaphoreType.DMA,
            pltpu.SemaphoreType.DMA,
        ],
        compiler_params=pltpu.CompilerParams(use_tc_tiling_on_sc=False),
    )(_emb_body)

    return gather(idx, weight)
